# Initial kernel scaffold; baseline (speedup 1.0000x reference)
import jax
import jax.numpy as jnp
from jax.experimental import pallas as pl


def kernel(x_node, x_edge, edge_index, W_edge, b_edge, W_node, b_node, gamma, beta):
    raise NotImplementedError("write your pallas kernel here")


# speedup vs baseline: 1.1519x; 1.1519x over previous
import functools

import jax
import jax.numpy as jnp
from jax import lax
from jax.experimental import pallas as pl
from jax.experimental.pallas import tpu as pltpu
from jax.experimental.pallas import tpu_sc as plsc

N = 10000
E = 320000
D_NODE = 128
D_EDGE = 16
D_MODEL = 128

_NC = 2
_NS = 16
_NW = _NC * _NS

_CHUNK = 128
_NCHUNKS = E // _CHUNK
_G = 4
_NGROUPS = _NCHUNKS // _G
_WORDS = D_MODEL // 2


def _node_proj_body(x_ref, w_ref, o_ref):
    acc = jnp.dot(x_ref[...], w_ref[...], preferred_element_type=jnp.float32)
    o_ref[...] = acc.astype(jnp.bfloat16)


def _node_proj(x_node, W_node):
    blk = 2000
    return pl.pallas_call(
        _node_proj_body,
        grid=(N // blk,),
        in_specs=[
            pl.BlockSpec((blk, D_NODE), lambda i: (i, 0)),
            pl.BlockSpec((D_NODE, D_MODEL), lambda i: (0, 0)),
        ],
        out_specs=pl.BlockSpec((blk, D_MODEL), lambda i: (i, 0)),
        out_shape=jax.ShapeDtypeStruct((N, D_MODEL), jnp.bfloat16),
    )(x_node, W_node)


def _gather_body(table_hbm, src_hbm, dst_hbm, s1_hbm, s2_hbm,
                 idx_s, idx_d, rows_s, rows_d, sem_i, sem_g, sem_w):
    wid = lax.axis_index("s") * _NC + lax.axis_index("c")
    nk = 19 + (wid < 17).astype(jnp.int32)

    def body(i, carry):
        grp = wid + i * _NW
        base = grp * _G
        pltpu.sync_copy(src_hbm.at[pl.ds(base, _G)], idx_s)
        pltpu.sync_copy(dst_hbm.at[pl.ds(base, _G)], idx_d)
        cps = [pltpu.async_copy(table_hbm.at[idx_s.at[j]], rows_s.at[j], sem_g)
               for j in range(_G)]
        cpd = [pltpu.async_copy(table_hbm.at[idx_d.at[j]], rows_d.at[j], sem_g)
               for j in range(_G)]
        for cp in cps + cpd:
            cp.wait()
        pltpu.sync_copy(rows_s, s1_hbm.at[pl.ds(base, _G)])
        pltpu.sync_copy(rows_d, s2_hbm.at[pl.ds(base, _G)])
        return carry

    lax.fori_loop(0, nk, body, 0)


_gather = functools.partial(
    pl.kernel,
    mesh=plsc.VectorSubcoreMesh(core_axis_name="c", subcore_axis_name="s"),
    compiler_params=pltpu.CompilerParams(use_tc_tiling_on_sc=False),
    out_type=[
        jax.ShapeDtypeStruct((_NCHUNKS, _CHUNK, _WORDS), jnp.int32),
        jax.ShapeDtypeStruct((_NCHUNKS, _CHUNK, _WORDS), jnp.int32),
    ],
    scratch_types=[
        pltpu.VMEM((_G, _CHUNK), jnp.int32),
        pltpu.VMEM((_G, _CHUNK), jnp.int32),
        pltpu.VMEM((_G, _CHUNK, _WORDS), jnp.int32),
        pltpu.VMEM((_G, _CHUNK, _WORDS), jnp.int32),
        pltpu.SemaphoreType.DMA,
        pltpu.SemaphoreType.DMA,
        pltpu.SemaphoreType.DMA,
    ],
)(_gather_body)


def _combine_body(xe_ref, s1_ref, s2_ref, we_ref, be_ref, bn_ref,
                  gamma_ref, beta_ref, o_ref):
    h = jnp.dot(xe_ref[...], we_ref[...], preferred_element_type=jnp.float32)
    s = (s1_ref[...].astype(jnp.float32) + s2_ref[...].astype(jnp.float32)) * 0.5
    h = h + s + be_ref[...] + bn_ref[...]
    mean = jnp.mean(h, axis=1, keepdims=True)
    d = h - mean
    var = jnp.mean(d * d, axis=1, keepdims=True)
    o_ref[...] = d * lax.rsqrt(var + 1e-5) * gamma_ref[...] + beta_ref[...]


def _combine(x_edge, s1, s2, W_edge, b_edge, b_node, gamma, beta):
    blk = 2000
    vec = pl.BlockSpec((1, D_MODEL), lambda i: (0, 0))
    return pl.pallas_call(
        _combine_body,
        grid=(E // blk,),
        in_specs=[
            pl.BlockSpec((blk, D_EDGE), lambda i: (i, 0)),
            pl.BlockSpec((blk, D_MODEL), lambda i: (i, 0)),
            pl.BlockSpec((blk, D_MODEL), lambda i: (i, 0)),
            pl.BlockSpec((D_EDGE, D_MODEL), lambda i: (0, 0)),
            vec, vec, vec, vec,
        ],
        out_specs=pl.BlockSpec((blk, D_MODEL), lambda i: (i, 0)),
        out_shape=jax.ShapeDtypeStruct((E, D_MODEL), jnp.float32),
    )(x_edge, s1, s2, W_edge, b_edge, b_node, gamma, beta)


def kernel(x_node, x_edge, edge_index, W_edge, b_edge, W_node, b_node, gamma, beta):
    g = _node_proj(x_node, W_node)
    table_i32 = lax.bitcast_convert_type(
        g.reshape(N, _WORDS, 2), jnp.int32)

    src2d = edge_index[0].reshape(_NCHUNKS, _CHUNK)
    dst2d = edge_index[1].reshape(_NCHUNKS, _CHUNK)
    s1_i32, s2_i32 = _gather(table_i32, src2d, dst2d)

    s1 = lax.bitcast_convert_type(s1_i32, jnp.bfloat16).reshape(E, D_MODEL)
    s2 = lax.bitcast_convert_type(s2_i32, jnp.bfloat16).reshape(E, D_MODEL)

    return _combine(x_edge, s1, s2, W_edge,
                    b_edge.reshape(1, D_MODEL), b_node.reshape(1, D_MODEL),
                    gamma.reshape(1, D_MODEL), beta.reshape(1, D_MODEL))


# speedup vs baseline: 3.3250x; 2.8865x over previous
import functools

import jax
import jax.numpy as jnp
from jax import lax
from jax.experimental import pallas as pl
from jax.experimental.pallas import tpu as pltpu
from jax.experimental.pallas import tpu_sc as plsc

N = 10000
E = 320000
D_NODE = 128
D_EDGE = 16
D_MODEL = 128

_NC = 2
_NS = 16
_NW = _NC * _NS

_CHUNK = 128
_NCHUNKS = E // _CHUNK
_G = 2
_NGROUPS = _NCHUNKS // _G
_BASE_ITERS = _NGROUPS // _NW
_EXTRA = _NGROUPS - _BASE_ITERS * _NW


def _node_proj_body(x_ref, w_ref, o_ref):
    o_ref[...] = jnp.dot(x_ref[...], w_ref[...],
                         preferred_element_type=jnp.float32)


def _node_proj(x_node, W_node):
    blk = 2000
    return pl.pallas_call(
        _node_proj_body,
        grid=(N // blk,),
        in_specs=[
            pl.BlockSpec((blk, D_NODE), lambda i: (i, 0)),
            pl.BlockSpec((D_NODE, D_MODEL), lambda i: (0, 0)),
        ],
        out_specs=pl.BlockSpec((blk, D_MODEL), lambda i: (i, 0)),
        out_shape=jax.ShapeDtypeStruct((N, D_MODEL), jnp.float32),
    )(x_node, W_node)


def _gather_body(table_hbm, src_hbm, dst_hbm, s1_hbm, s2_hbm,
                 idx_s, idx_d, rows_s, rows_d, sem_g):
    wid = lax.axis_index("s") * _NC + lax.axis_index("c")
    nk = _BASE_ITERS + (wid < _EXTRA).astype(jnp.int32)

    def body(i, carry):
        grp = wid + i * _NW
        base = grp * _G
        pltpu.sync_copy(src_hbm.at[pl.ds(base, _G)], idx_s)
        pltpu.sync_copy(dst_hbm.at[pl.ds(base, _G)], idx_d)
        cps = [pltpu.async_copy(table_hbm.at[idx_s.at[j]], rows_s.at[j], sem_g)
               for j in range(_G)]
        cpd = [pltpu.async_copy(table_hbm.at[idx_d.at[j]], rows_d.at[j], sem_g)
               for j in range(_G)]
        for cp in cps + cpd:
            cp.wait()
        pltpu.sync_copy(rows_s, s1_hbm.at[pl.ds(base, _G)])
        pltpu.sync_copy(rows_d, s2_hbm.at[pl.ds(base, _G)])
        return carry

    lax.fori_loop(0, nk, body, 0)


_gather = functools.partial(
    pl.kernel,
    mesh=plsc.VectorSubcoreMesh(core_axis_name="c", subcore_axis_name="s"),
    out_type=[
        jax.ShapeDtypeStruct((_NCHUNKS, _CHUNK, D_MODEL), jnp.float32),
        jax.ShapeDtypeStruct((_NCHUNKS, _CHUNK, D_MODEL), jnp.float32),
    ],
    scratch_types=[
        pltpu.VMEM((_G, _CHUNK), jnp.int32),
        pltpu.VMEM((_G, _CHUNK), jnp.int32),
        pltpu.VMEM((_G, _CHUNK, D_MODEL), jnp.float32),
        pltpu.VMEM((_G, _CHUNK, D_MODEL), jnp.float32),
        pltpu.SemaphoreType.DMA,
    ],
)(_gather_body)


def _combine_body(xe_ref, s1_ref, s2_ref, we_ref, be_ref, bn_ref,
                  gamma_ref, beta_ref, o_ref):
    h = jnp.dot(xe_ref[...], we_ref[...], preferred_element_type=jnp.float32)
    s = (s1_ref[...] + s2_ref[...]) * 0.5
    h = h + s + be_ref[...] + bn_ref[...]
    mean = jnp.mean(h, axis=1, keepdims=True)
    d = h - mean
    var = jnp.mean(d * d, axis=1, keepdims=True)
    o_ref[...] = d * lax.rsqrt(var + 1e-5) * gamma_ref[...] + beta_ref[...]


def _combine(x_edge, s1, s2, W_edge, b_edge, b_node, gamma, beta):
    blk = 2000
    vec = pl.BlockSpec((1, D_MODEL), lambda i: (0, 0))
    return pl.pallas_call(
        _combine_body,
        grid=(E // blk,),
        in_specs=[
            pl.BlockSpec((blk, D_EDGE), lambda i: (i, 0)),
            pl.BlockSpec((blk, D_MODEL), lambda i: (i, 0)),
            pl.BlockSpec((blk, D_MODEL), lambda i: (i, 0)),
            pl.BlockSpec((D_EDGE, D_MODEL), lambda i: (0, 0)),
            vec, vec, vec, vec,
        ],
        out_specs=pl.BlockSpec((blk, D_MODEL), lambda i: (i, 0)),
        out_shape=jax.ShapeDtypeStruct((E, D_MODEL), jnp.float32),
    )(x_edge, s1, s2, W_edge, b_edge, b_node, gamma, beta)


def kernel(x_node, x_edge, edge_index, W_edge, b_edge, W_node, b_node, gamma, beta):
    table = _node_proj(x_node, W_node)

    src2d = edge_index[0].reshape(_NCHUNKS, _CHUNK)
    dst2d = edge_index[1].reshape(_NCHUNKS, _CHUNK)
    s1_3d, s2_3d = _gather(table, src2d, dst2d)

    s1 = s1_3d.reshape(E, D_MODEL)
    s2 = s2_3d.reshape(E, D_MODEL)

    return _combine(x_edge, s1, s2, W_edge,
                    b_edge.reshape(1, D_MODEL), b_node.reshape(1, D_MODEL),
                    gamma.reshape(1, D_MODEL), beta.reshape(1, D_MODEL))


# speedup vs baseline: 3.8899x; 1.1699x over previous
import functools

import jax
import jax.numpy as jnp
from jax import lax
from jax.experimental import pallas as pl
from jax.experimental.pallas import tpu as pltpu
from jax.experimental.pallas import tpu_sc as plsc

N = 10000
E = 320000
D_NODE = 128
D_EDGE = 16
D_MODEL = 128

_NC = 2
_NS = 16
_NW = _NC * _NS

_CHUNK = 128
_NCHUNKS = E // _CHUNK
_BASE_ITERS = _NCHUNKS // _NW
_EXTRA = _NCHUNKS - _BASE_ITERS * _NW
_LANES = 16


def _node_proj_body(x_ref, w_ref, o_ref):
    o_ref[...] = jnp.dot(x_ref[...], w_ref[...],
                         preferred_element_type=jnp.float32)


def _node_proj(x_node, W_node):
    blk = 2000
    return pl.pallas_call(
        _node_proj_body,
        grid=(N // blk,),
        in_specs=[
            pl.BlockSpec((blk, D_NODE), lambda i: (i, 0)),
            pl.BlockSpec((D_NODE, D_MODEL), lambda i: (0, 0)),
        ],
        out_specs=pl.BlockSpec((blk, D_MODEL), lambda i: (i, 0)),
        out_shape=jax.ShapeDtypeStruct((N, D_MODEL), jnp.float32),
    )(x_node, W_node)


def _gather_body(table_hbm, src_hbm, dst_hbm, s_hbm,
                 idx_s0, idx_d0, rs0, rd0, o0,
                 idx_s1, idx_d1, rs1, rd1, o1,
                 gsem0, gsem1, wsem0, wsem1):
    wid = lax.axis_index("s") * _NC + lax.axis_index("c")
    nk = _BASE_ITERS + (wid < _EXTRA).astype(jnp.int32)
    npairs = (nk + 1) // 2

    slots = (
        (idx_s0, idx_d0, rs0, rd0, o0, gsem0, wsem0),
        (idx_s1, idx_d1, rs1, rd1, o1, gsem1, wsem1),
    )

    def body(i, carry):
        for slot in (0, 1):
            idx_s, idx_d, rs, rd, o, gsem, wsem = slots[slot]
            k = 2 * i + slot

            @pl.when(k < nk)
            def _start():
                grp = wid + k * _NW
                pltpu.sync_copy(src_hbm.at[grp], idx_s)
                pltpu.sync_copy(dst_hbm.at[grp], idx_d)
                pltpu.async_copy(table_hbm.at[idx_s], rs, gsem)
                pltpu.async_copy(table_hbm.at[idx_d], rd, gsem)

        for slot in (0, 1):
            idx_s, idx_d, rs, rd, o, gsem, wsem = slots[slot]
            k = 2 * i + slot

            @pl.when(k < nk)
            def _process():
                grp = wid + k * _NW
                pltpu.make_async_copy(table_hbm.at[idx_s], rs, gsem).wait()
                pltpu.make_async_copy(table_hbm.at[idx_d], rd, gsem).wait()

                @pl.when(i > 0)
                def _drain():
                    pltpu.make_async_copy(s_hbm.at[0], o, wsem).wait()

                def addrow(r, c):
                    for j in range(D_MODEL // _LANES):
                        sl = pl.ds(j * _LANES, _LANES)
                        o[r, sl] = rs[r, sl] + rd[r, sl]
                    return c

                lax.fori_loop(0, _CHUNK, addrow, 0)
                pltpu.async_copy(o, s_hbm.at[grp], wsem)

        return carry

    lax.fori_loop(0, npairs, body, 0)

    pltpu.make_async_copy(s_hbm.at[0], o0, wsem0).wait()
    pltpu.make_async_copy(s_hbm.at[0], o1, wsem1).wait()


_gather = functools.partial(
    pl.kernel,
    mesh=plsc.VectorSubcoreMesh(core_axis_name="c", subcore_axis_name="s"),
    out_type=jax.ShapeDtypeStruct((_NCHUNKS, _CHUNK, D_MODEL), jnp.float32),
    scratch_types=[
        pltpu.VMEM((_CHUNK,), jnp.int32),
        pltpu.VMEM((_CHUNK,), jnp.int32),
        pltpu.VMEM((_CHUNK, D_MODEL), jnp.float32),
        pltpu.VMEM((_CHUNK, D_MODEL), jnp.float32),
        pltpu.VMEM((_CHUNK, D_MODEL), jnp.float32),
        pltpu.VMEM((_CHUNK,), jnp.int32),
        pltpu.VMEM((_CHUNK,), jnp.int32),
        pltpu.VMEM((_CHUNK, D_MODEL), jnp.float32),
        pltpu.VMEM((_CHUNK, D_MODEL), jnp.float32),
        pltpu.VMEM((_CHUNK, D_MODEL), jnp.float32),
        pltpu.SemaphoreType.DMA,
        pltpu.SemaphoreType.DMA,
        pltpu.SemaphoreType.DMA,
        pltpu.SemaphoreType.DMA,
    ],
)(_gather_body)


def _combine_body(xe_ref, s_ref, we_ref, be_ref, bn_ref,
                  gamma_ref, beta_ref, o_ref):
    h = jnp.dot(xe_ref[...], we_ref[...], preferred_element_type=jnp.float32)
    h = h + s_ref[...] * 0.5 + be_ref[...] + bn_ref[...]
    mean = jnp.mean(h, axis=1, keepdims=True)
    d = h - mean
    var = jnp.mean(d * d, axis=1, keepdims=True)
    o_ref[...] = d * lax.rsqrt(var + 1e-5) * gamma_ref[...] + beta_ref[...]


def _combine(x_edge, s, W_edge, b_edge, b_node, gamma, beta):
    blk = 2000
    vec = pl.BlockSpec((1, D_MODEL), lambda i: (0, 0))
    return pl.pallas_call(
        _combine_body,
        grid=(E // blk,),
        in_specs=[
            pl.BlockSpec((blk, D_EDGE), lambda i: (i, 0)),
            pl.BlockSpec((blk, D_MODEL), lambda i: (i, 0)),
            pl.BlockSpec((D_EDGE, D_MODEL), lambda i: (0, 0)),
            vec, vec, vec, vec,
        ],
        out_specs=pl.BlockSpec((blk, D_MODEL), lambda i: (i, 0)),
        out_shape=jax.ShapeDtypeStruct((E, D_MODEL), jnp.float32),
    )(x_edge, s, W_edge, b_edge, b_node, gamma, beta)


def kernel(x_node, x_edge, edge_index, W_edge, b_edge, W_node, b_node, gamma, beta):
    table = _node_proj(x_node, W_node)

    src2d = edge_index[0].reshape(_NCHUNKS, _CHUNK)
    dst2d = edge_index[1].reshape(_NCHUNKS, _CHUNK)
    s = _gather(table, src2d, dst2d).reshape(E, D_MODEL)

    return _combine(x_edge, s, W_edge,
                    b_edge.reshape(1, D_MODEL), b_node.reshape(1, D_MODEL),
                    gamma.reshape(1, D_MODEL), beta.reshape(1, D_MODEL))
